# baseline (speedup 1.0000x reference)
import functools

import jax
import jax.numpy as jnp
from jax import lax
from jax.experimental import pallas as pl
from jax.experimental.pallas import tpu as pltpu
from jax.experimental.pallas import tpu_sc as plsc

N = 10000
E = 160000
D = 512
NCHUNK = 4
CW = 128
NC = 2
NS = 16
W = 80
EPS = E // NS
WROWS = 128
SROWS = 32
PADE = WROWS * W
NPAD = 10240
STRIPE = NPAD // NS

_sc_mesh = plsc.VectorSubcoreMesh(
    core_axis_name="c", subcore_axis_name="s", num_cores=NC, num_subcores=NS
)


@functools.partial(
    pl.kernel,
    out_type=jax.ShapeDtypeStruct((NCHUNK * NPAD, CW), jnp.float32),
    mesh=_sc_mesh,
    scratch_types=[
        pltpu.VMEM((SROWS, W), jnp.int32),
        pltpu.VMEM((SROWS, W), jnp.int32),
        pltpu.VMEM((W, CW), jnp.float32),
        pltpu.VMEM((W, CW), jnp.float32),
        pltpu.VMEM((W, CW), jnp.float32),
        pltpu.VMEM_SHARED((NPAD, CW), jnp.float32),
        pltpu.SemaphoreType.DMA,
        pltpu.SemaphoreType.DMA,
        pltpu.SemaphoreType.DMA,
        pltpu.SemaphoreType.DMA,
        pltpu.SemaphoreType.DMA,
        pltpu.SemaphoreType.DMA,
    ],
    cost_estimate=pl.CostEstimate(
        flops=0, bytes_accessed=700_000_000, transcendentals=0),
)
def _sc_segment_sum(h_hbm, gidx_hbm, sidx_hbm, out_hbm,
                    gi_v, si_v, rows_a, rows_b, rows_c, acc_sh,
                    sema, semb, semc, ssema, ssemb, ssemc):
    ci = lax.axis_index("c")
    sid = lax.axis_index("s")

    def gather(idx_row, buf, sem):
        pltpu.async_copy(h_hbm.at[idx_row], buf, sem)

    def drain(buf, sem):
        pltpu.make_async_copy(h_hbm.at[gi_v.at[0]], buf, sem).wait()

    def scatter(buf, idx_row):
        pltpu.sync_copy(buf, acc_sh.at[idx_row], add=True)

    def ascatter(buf, idx_row, sem):
        pltpu.async_copy(buf, acc_sh.at[idx_row], sem, add=True)

    for cc in range(NCHUNK // NC):
        c = ci * (NCHUNK // NC) + cc

        @pl.loop(0, W)
        def _(r):
            @pl.loop(0, CW, step=16)
            def _(i):
                rows_a[r, pl.ds(i, 16)] = jnp.zeros((16,), jnp.float32)

        r0 = sid * STRIPE
        for k in range(STRIPE // W):
            pltpu.sync_copy(rows_a, acc_sh.at[pl.ds(r0 + k * W, W)])

        plsc.subcore_barrier()

        @pl.loop(0, WROWS // SROWS)
        def _(st):
            pltpu.sync_copy(
                gidx_hbm.at[c * NS + sid].at[pl.ds(st * SROWS, SROWS)], gi_v)
            pltpu.sync_copy(
                sidx_hbm.at[sid].at[pl.ds(st * SROWS, SROWS)], si_v)

            gather(gi_v.at[0], rows_a, sema)
            gather(gi_v.at[1], rows_b, semb)

            @pl.loop(0, (SROWS - 2) // 3)
            def _(i):
                w = i * 3
                gather(gi_v.at[w + 2], rows_c, semc)
                drain(rows_a, sema)
                ascatter(rows_a, si_v.at[w], ssema)
                drain(rows_b, semb)
                ascatter(rows_b, si_v.at[w + 1], ssemb)
                drain(rows_a, ssema)
                gather(gi_v.at[w + 3], rows_a, sema)
                drain(rows_c, semc)
                ascatter(rows_c, si_v.at[w + 2], ssemc)
                drain(rows_b, ssemb)
                gather(gi_v.at[w + 4], rows_b, semb)
                drain(rows_c, ssemc)

            drain(rows_a, sema)
            scatter(rows_a, si_v.at[SROWS - 2])
            drain(rows_b, semb)
            scatter(rows_b, si_v.at[SROWS - 1])

        plsc.subcore_barrier()

        pltpu.sync_copy(acc_sh.at[pl.ds(r0, STRIPE)],
                        out_hbm.at[pl.ds(c * NPAD + r0, STRIPE)])

        if cc + 1 < NCHUNK // NC:
            plsc.subcore_barrier()


MB = 1000


def _layer_dot(agg_ref, h_ref, wr_ref, wo_ref, br_ref):
    acc = jnp.dot(h_ref[...], wo_ref[...], preferred_element_type=jnp.float32)
    for c in range(NCHUNK):
        acc += jnp.dot(agg_ref[c], wr_ref[pl.ds(c * CW, CW), :],
                       preferred_element_type=jnp.float32)
    return jnp.maximum(acc + br_ref[...], 0.0)


def _layer_body(agg_ref, h_ref, wr_ref, wo_ref, br_ref, out_ref):
    out_ref[...] = _layer_dot(agg_ref, h_ref, wr_ref, wo_ref, br_ref)


def _layer_pool_body(agg_ref, h_ref, wr_ref, wo_ref, br_ref, out_ref, pool_ref):
    acc = _layer_dot(agg_ref, h_ref, wr_ref, wo_ref, br_ref)
    out_ref[...] = acc

    @pl.when(pl.program_id(0) == 0)
    def _():
        pool_ref[...] = jnp.zeros_like(pool_ref)

    pool_ref[...] += jnp.sum(acc, axis=0, keepdims=True)


def _tc_layer(agg3, h, Wr, br, Wo, with_pool):
    in_specs = [
        pl.BlockSpec((NCHUNK, MB, CW), lambda m: (0, m, 0)),
        pl.BlockSpec((MB, D), lambda m: (m, 0)),
        pl.BlockSpec((D, D), lambda m: (0, 0)),
        pl.BlockSpec((D, D), lambda m: (0, 0)),
        pl.BlockSpec((1, D), lambda m: (0, 0)),
    ]
    if with_pool:
        return pl.pallas_call(
            _layer_pool_body,
            grid=(N // MB,),
            in_specs=in_specs,
            out_specs=[pl.BlockSpec((MB, D), lambda m: (m, 0)),
                       pl.BlockSpec((1, D), lambda m: (0, 0))],
            out_shape=[jax.ShapeDtypeStruct((N, D), jnp.float32),
                       jax.ShapeDtypeStruct((1, D), jnp.float32)],
        )(agg3, h, Wr, Wo, br.reshape(1, D))
    return pl.pallas_call(
        _layer_body,
        grid=(N // MB,),
        in_specs=in_specs,
        out_specs=pl.BlockSpec((MB, D), lambda m: (m, 0)),
        out_shape=jax.ShapeDtypeStruct((N, D), jnp.float32),
    )(agg3, h, Wr, Wo, br.reshape(1, D))


def _head_body(p_ref, w6_ref, b6_ref, w7_ref, b7_ref, o_ref):
    g = jnp.maximum(
        jnp.dot(p_ref[...], w6_ref[...], preferred_element_type=jnp.float32)
        + b6_ref[...], 0.0)
    o_ref[...] = (jnp.dot(g, w7_ref[...], preferred_element_type=jnp.float32)
                  + b7_ref[...])


def _tc_head(pooled, W6, b6, W7, b7):
    return pl.pallas_call(
        _head_body,
        out_shape=jax.ShapeDtypeStruct((1, D), jnp.float32),
    )(pooled, W6, b6.reshape(1, D), W7, b7.reshape(1, D))


def kernel(x, edge_index, Wr1, br1, Wo1, Wr2, br2, Wo2, Wr3, br3, Wo3,
           Wr4, br4, Wo4, Wr5, br5, Wo5, W6, b6, W7, b7):
    src = edge_index[0]
    dst = edge_index[1]

    npadE = PADE - EPS
    pad_rows = (jnp.arange(npadE, dtype=jnp.int32) * 977) % N
    srcp = jnp.concatenate(
        [src.reshape(NS, EPS),
         jnp.broadcast_to(pad_rows, (NS, npadE))], axis=1)
    gidx = (srcp[None] * NCHUNK
            + jnp.arange(NCHUNK, dtype=jnp.int32)[:, None, None])
    gidx = gidx.reshape(NCHUNK * NS, WROWS, W)

    pad_dst = (NPAD - 8) + (jnp.arange(npadE, dtype=jnp.int32) % 8)
    sidx = jnp.concatenate(
        [dst.reshape(NS, EPS),
         jnp.broadcast_to(pad_dst, (NS, npadE))], axis=1)
    sidx = sidx.reshape(NS, WROWS, W)

    layers = ((Wr1, br1, Wo1), (Wr2, br2, Wo2), (Wr3, br3, Wo3),
              (Wr4, br4, Wo4), (Wr5, br5, Wo5))

    h = x
    pooled = None
    for li, (Wr, br, Wo) in enumerate(layers):
        agg = _sc_segment_sum(h.reshape(NCHUNK * N, CW), gidx, sidx)
        agg3 = agg.reshape(NCHUNK, NPAD, CW)
        if li == len(layers) - 1:
            h, pooled = _tc_layer(agg3, h, Wr, br, Wo, with_pool=True)
        else:
            h = _tc_layer(agg3, h, Wr, br, Wo, with_pool=False)

    return _tc_head(pooled, W6, b6, W7, b7)


# speedup vs baseline: 1.2000x; 1.2000x over previous
import functools

import jax
import jax.numpy as jnp
from jax import lax
from jax.experimental import pallas as pl
from jax.experimental.pallas import tpu as pltpu
from jax.experimental.pallas import tpu_sc as plsc

N = 10000
E = 160000
D = 512
NCHUNK = 4
CW = 128
NC = 2
NS = 16
W = 80
EPS = E // NS
WROWS = 128
SROWS = 32
PADE = WROWS * W
NPAD = 10240
STRIPE = NPAD // NS

_sc_mesh = plsc.VectorSubcoreMesh(
    core_axis_name="c", subcore_axis_name="s", num_cores=NC, num_subcores=NS
)


@functools.partial(
    pl.kernel,
    out_type=jax.ShapeDtypeStruct((NCHUNK * NPAD, CW), jnp.float32),
    mesh=_sc_mesh,
    scratch_types=[
        pltpu.VMEM((SROWS, W), jnp.int32),
        pltpu.VMEM((SROWS, W), jnp.int32),
        pltpu.VMEM((W, CW), jnp.float32),
        pltpu.VMEM((W, CW), jnp.float32),
        pltpu.VMEM((W, CW), jnp.float32),
        pltpu.VMEM_SHARED((NPAD, CW), jnp.float32),
        pltpu.SemaphoreType.DMA,
        pltpu.SemaphoreType.DMA,
        pltpu.SemaphoreType.DMA,
    ],
    cost_estimate=pl.CostEstimate(
        flops=0, bytes_accessed=700_000_000, transcendentals=0),
)
def _sc_segment_sum(h_hbm, gidx_hbm, sidx_hbm, out_hbm,
                    gi_v, si_v, rows_a, rows_b, rows_c, acc_sh,
                    sema, semb, semc):
    ci = lax.axis_index("c")
    sid = lax.axis_index("s")

    def gather(idx_row, buf, sem):
        pltpu.async_copy(h_hbm.at[idx_row], buf, sem)

    def drain(buf, sem):
        pltpu.make_async_copy(h_hbm.at[gi_v.at[0]], buf, sem).wait()

    def scatter(buf, idx_row):
        pltpu.sync_copy(buf, acc_sh.at[idx_row], add=True)

    for cc in range(NCHUNK // NC):
        c = ci * (NCHUNK // NC) + cc

        @pl.loop(0, W)
        def _(r):
            @pl.loop(0, CW, step=16)
            def _(i):
                rows_a[r, pl.ds(i, 16)] = jnp.zeros((16,), jnp.float32)

        r0 = sid * STRIPE
        for k in range(STRIPE // W):
            pltpu.sync_copy(rows_a, acc_sh.at[pl.ds(r0 + k * W, W)])

        plsc.subcore_barrier()

        @pl.loop(0, WROWS // SROWS)
        def _(st):
            pltpu.sync_copy(
                gidx_hbm.at[c * NS + sid].at[pl.ds(st * SROWS, SROWS)], gi_v)
            pltpu.sync_copy(
                sidx_hbm.at[sid].at[pl.ds(st * SROWS, SROWS)], si_v)

            gather(gi_v.at[0], rows_a, sema)
            gather(gi_v.at[1], rows_b, semb)

            @pl.loop(0, (SROWS - 2) // 3)
            def _(i):
                w = i * 3
                gather(gi_v.at[w + 2], rows_c, semc)
                drain(rows_a, sema)
                scatter(rows_a, si_v.at[w])
                gather(gi_v.at[w + 3], rows_a, sema)
                drain(rows_b, semb)
                scatter(rows_b, si_v.at[w + 1])
                gather(gi_v.at[w + 4], rows_b, semb)
                drain(rows_c, semc)
                scatter(rows_c, si_v.at[w + 2])

            drain(rows_a, sema)
            scatter(rows_a, si_v.at[SROWS - 2])
            drain(rows_b, semb)
            scatter(rows_b, si_v.at[SROWS - 1])

        plsc.subcore_barrier()

        pltpu.sync_copy(acc_sh.at[pl.ds(r0, STRIPE)],
                        out_hbm.at[pl.ds(c * NPAD + r0, STRIPE)])

        if cc + 1 < NCHUNK // NC:
            plsc.subcore_barrier()


MB = 1000


def _layer_dot(agg_ref, h_ref, wr_ref, wo_ref, br_ref):
    acc = jnp.dot(h_ref[...], wo_ref[...], preferred_element_type=jnp.float32)
    for c in range(NCHUNK):
        acc += jnp.dot(agg_ref[c], wr_ref[pl.ds(c * CW, CW), :],
                       preferred_element_type=jnp.float32)
    return jnp.maximum(acc + br_ref[...], 0.0)


def _layer_body(agg_ref, h_ref, wr_ref, wo_ref, br_ref, out_ref):
    out_ref[...] = _layer_dot(agg_ref, h_ref, wr_ref, wo_ref, br_ref)


def _layer_pool_body(agg_ref, h_ref, wr_ref, wo_ref, br_ref, out_ref, pool_ref):
    acc = _layer_dot(agg_ref, h_ref, wr_ref, wo_ref, br_ref)
    out_ref[...] = acc

    @pl.when(pl.program_id(0) == 0)
    def _():
        pool_ref[...] = jnp.zeros_like(pool_ref)

    pool_ref[...] += jnp.sum(acc, axis=0, keepdims=True)


def _tc_layer(agg3, h, Wr, br, Wo, with_pool):
    in_specs = [
        pl.BlockSpec((NCHUNK, MB, CW), lambda m: (0, m, 0)),
        pl.BlockSpec((MB, D), lambda m: (m, 0)),
        pl.BlockSpec((D, D), lambda m: (0, 0)),
        pl.BlockSpec((D, D), lambda m: (0, 0)),
        pl.BlockSpec((1, D), lambda m: (0, 0)),
    ]
    if with_pool:
        return pl.pallas_call(
            _layer_pool_body,
            grid=(N // MB,),
            in_specs=in_specs,
            out_specs=[pl.BlockSpec((MB, D), lambda m: (m, 0)),
                       pl.BlockSpec((1, D), lambda m: (0, 0))],
            out_shape=[jax.ShapeDtypeStruct((N, D), jnp.float32),
                       jax.ShapeDtypeStruct((1, D), jnp.float32)],
        )(agg3, h, Wr, Wo, br.reshape(1, D))
    return pl.pallas_call(
        _layer_body,
        grid=(N // MB,),
        in_specs=in_specs,
        out_specs=pl.BlockSpec((MB, D), lambda m: (m, 0)),
        out_shape=jax.ShapeDtypeStruct((N, D), jnp.float32),
    )(agg3, h, Wr, Wo, br.reshape(1, D))


def _head_body(p_ref, w6_ref, b6_ref, w7_ref, b7_ref, o_ref):
    g = jnp.maximum(
        jnp.dot(p_ref[...], w6_ref[...], preferred_element_type=jnp.float32)
        + b6_ref[...], 0.0)
    o_ref[...] = (jnp.dot(g, w7_ref[...], preferred_element_type=jnp.float32)
                  + b7_ref[...])


def _tc_head(pooled, W6, b6, W7, b7):
    return pl.pallas_call(
        _head_body,
        out_shape=jax.ShapeDtypeStruct((1, D), jnp.float32),
    )(pooled, W6, b6.reshape(1, D), W7, b7.reshape(1, D))


def kernel(x, edge_index, Wr1, br1, Wo1, Wr2, br2, Wo2, Wr3, br3, Wo3,
           Wr4, br4, Wo4, Wr5, br5, Wo5, W6, b6, W7, b7):
    src = edge_index[0]
    dst = edge_index[1]

    npadE = PADE - EPS
    pad_rows = (jnp.arange(npadE, dtype=jnp.int32) * 977) % N
    srcp = jnp.concatenate(
        [src.reshape(NS, EPS),
         jnp.broadcast_to(pad_rows, (NS, npadE))], axis=1)
    gidx = (srcp[None] * NCHUNK
            + jnp.arange(NCHUNK, dtype=jnp.int32)[:, None, None])
    gidx = gidx.reshape(NCHUNK * NS, WROWS, W)

    pad_dst = (NPAD - 8) + (jnp.arange(npadE, dtype=jnp.int32) % 8)
    sidx = jnp.concatenate(
        [dst.reshape(NS, EPS),
         jnp.broadcast_to(pad_dst, (NS, npadE))], axis=1)
    sidx = sidx.reshape(NS, WROWS, W)

    layers = ((Wr1, br1, Wo1), (Wr2, br2, Wo2), (Wr3, br3, Wo3),
              (Wr4, br4, Wo4), (Wr5, br5, Wo5))

    h = x
    pooled = None
    for li, (Wr, br, Wo) in enumerate(layers):
        agg = _sc_segment_sum(h.reshape(NCHUNK * N, CW), gidx, sidx)
        agg3 = agg.reshape(NCHUNK, NPAD, CW)
        if li == len(layers) - 1:
            h, pooled = _tc_layer(agg3, h, Wr, br, Wo, with_pool=True)
        else:
            h = _tc_layer(agg3, h, Wr, br, Wo, with_pool=False)

    return _tc_head(pooled, W6, b6, W7, b7)
